# baseline (speedup 1.0000x reference)
import functools
import math

import jax
import jax.numpy as jnp
from jax import lax
from jax.experimental import pallas as pl
from jax.experimental.pallas import tpu as pltpu
from jax.experimental.pallas import tpu_sc as plsc

_NC = 2
_NS = 16
_NW = _NC * _NS

_BB = 128
_L = 16


def _build_lookup(batch: int, seq: int, dim: int, scale: float):
    assert batch == _NW * _BB and dim % 8 == 0 and seq % 2 == 0
    fb_n = dim // 8

    mesh = plsc.VectorSubcoreMesh(core_axis_name="c", subcore_axis_name="s",
                                  num_cores=_NC, num_subcores=_NS)

    @functools.partial(
        pl.kernel,
        mesh=mesh,
        compiler_params=pltpu.CompilerParams(use_tc_tiling_on_sc=False,
                                             needs_layout_passes=False),
        out_type=jax.ShapeDtypeStruct((seq, fb_n, _NW, 8, 128), jnp.float32),
        scratch_types=[
            pltpu.VMEM((seq, _BB), jnp.int32),
            pltpu.VMEM((_BB, dim // 2), jnp.int32),
            pltpu.VMEM((_BB, dim // 2), jnp.int32),
            pltpu.VMEM((dim, _BB + 1), jnp.float32),
            pltpu.VMEM((dim, _BB + 1), jnp.float32),
            pltpu.SemaphoreType.DMA,
            pltpu.SemaphoreType.DMA,
            pltpu.SemaphoreType.DMA,
            pltpu.SemaphoreType.DMA,
        ],
    )
    def k(table_hbm, idx_hbm, out_hbm, idx_v, buf0, buf1, tb0, tb1,
          g0, g1, s0, s1):
        wid = lax.axis_index("s") * _NC + lax.axis_index("c")
        bufs, tbs, gsems, ssems = (buf0, buf1), (tb0, tb1), (g0, g1), (s0, s1)

        pltpu.sync_copy(idx_hbm.at[:, pl.ds(wid * _BB, _BB)], idx_v)

        def start_gather(s, p):
            pltpu.async_copy(table_hbm.at[idx_v.at[s]], bufs[p], gsems[p])

        def wait_gather(p):
            pltpu.make_async_copy(table_hbm.at[pl.ds(0, _BB)], bufs[p],
                                  gsems[p]).wait()

        def wait_stores(p):
            pltpu.make_async_copy(table_hbm.at[pl.ds(0, _BB)], bufs[p],
                                  ssems[p]).wait()
            pltpu.make_async_copy(table_hbm.at[pl.ds(0, _BB)], bufs[p],
                                  ssems[p]).wait()

        iota = lax.iota(jnp.int32, _L)
        rows_f0 = [iota + f0 * _L for f0 in range(dim // _L)]

        def transpose_scale(p):
            buf, tb = bufs[p], tbs[p]

            @plsc.parallel_loop(0, _BB, unroll=4)
            def _(t):
                cols = jnp.full((_L,), 0, jnp.int32) + t
                for f0 in range(dim // (2 * _L)):
                    w = buf[t, pl.ds(f0 * _L, _L)]
                    ab = plsc.bitcast(w, jnp.bfloat16)
                    a, b = plsc.unpack(ab, format=plsc.PackFormat.INTERLEAVED,
                                       preferred_element_type=jnp.float32)
                    plsc.store_scatter(tb, [rows_f0[f0], cols], a * scale)
                    plsc.store_scatter(tb, [rows_f0[f0 + dim // (2 * _L)],
                                            cols], b * scale)

        def start_stores(s, p):
            tb = tbs[p]
            for fb in range(fb_n):
                pltpu.async_copy(tb.at[pl.ds(fb * 8, 8), pl.ds(0, _BB)],
                                 out_hbm.at[s, fb, wid], ssems[p])

        start_gather(0, 0)
        start_gather(1, 1)

        def loop_body(s2, carry):
            for p in range(2):
                s = 2 * s2 + p
                wait_gather(p)

                @pl.when(s2 >= 1)
                def _():
                    wait_stores(p)

                transpose_scale(p)
                start_stores(s, p)

                @pl.when(s + 2 < seq)
                def _():
                    start_gather(s + 2, p)
            return carry

        lax.fori_loop(0, seq // 2, loop_body, 0)
        wait_stores(0)
        wait_stores(1)

    return k


def kernel(token_ids_batch, table):
    batch, seq = token_ids_batch.shape
    vocab, dim = table.shape
    scale = math.sqrt(dim)
    idx_t = jnp.transpose(token_ids_batch).astype(jnp.int32)
    t16 = table.astype(jnp.bfloat16)
    tshuf = jnp.stack([t16[:, :dim // 2], t16[:, dim // 2:]], axis=-1)
    ti32 = jax.lax.bitcast_convert_type(tshuf, jnp.int32)
    out5d = _build_lookup(batch, seq, dim, scale)(ti32, idx_t)
    out = jnp.transpose(out5d, (2, 4, 0, 1, 3)).reshape(batch, seq, dim)
    return out


# speedup vs baseline: 1.2859x; 1.2859x over previous
import functools
import math

import jax
import jax.numpy as jnp
from jax import lax
from jax.experimental import pallas as pl
from jax.experimental.pallas import tpu as pltpu
from jax.experimental.pallas import tpu_sc as plsc

_NC = 2
_NS = 16
_NW = _NC * _NS

_BB = 128
_L = 16


def _build_lookup(batch: int, seq: int, dim: int, scale: float):
    assert batch == _NW * _BB and dim % 8 == 0 and seq % 2 == 0
    fb_n = dim // 8

    mesh = plsc.VectorSubcoreMesh(core_axis_name="c", subcore_axis_name="s",
                                  num_cores=_NC, num_subcores=_NS)

    @functools.partial(
        pl.kernel,
        mesh=mesh,
        compiler_params=pltpu.CompilerParams(use_tc_tiling_on_sc=False,
                                             needs_layout_passes=False),
        out_type=jax.ShapeDtypeStruct((seq, fb_n, _NW, 8, 128), jnp.float32),
        scratch_types=[
            pltpu.VMEM((seq, _BB), jnp.int32),
            pltpu.VMEM((_BB, dim), jnp.float32),
            pltpu.VMEM((_BB, dim), jnp.float32),
            pltpu.VMEM((dim, _BB + 1), jnp.float32),
            pltpu.VMEM((dim, _BB + 1), jnp.float32),
            pltpu.SemaphoreType.DMA,
            pltpu.SemaphoreType.DMA,
            pltpu.SemaphoreType.DMA,
            pltpu.SemaphoreType.DMA,
        ],
    )
    def k(table_hbm, idx_hbm, out_hbm, idx_v, buf0, buf1, tb0, tb1,
          g0, g1, s0, s1):
        wid = lax.axis_index("s") * _NC + lax.axis_index("c")
        bufs, tbs, gsems, ssems = (buf0, buf1), (tb0, tb1), (g0, g1), (s0, s1)

        pltpu.sync_copy(idx_hbm.at[:, pl.ds(wid * _BB, _BB)], idx_v)

        def start_gather(s, p):
            pltpu.async_copy(table_hbm.at[idx_v.at[s]], bufs[p], gsems[p])

        def wait_gather(p):
            pltpu.make_async_copy(table_hbm.at[pl.ds(0, _BB)], bufs[p],
                                  gsems[p]).wait()

        def wait_stores(p):
            pltpu.make_async_copy(table_hbm.at[pl.ds(0, _BB)], bufs[p],
                                  ssems[p]).wait()

        iota = lax.iota(jnp.int32, _L)
        rows_f0 = [iota + f0 * _L for f0 in range(dim // _L)]

        def transpose_scale(p):
            buf, tb = bufs[p], tbs[p]

            @plsc.parallel_loop(0, _BB, unroll=4)
            def _(t):
                cols = jnp.full((_L,), 0, jnp.int32) + t
                for f0 in range(dim // _L):
                    v = buf[t, pl.ds(f0 * _L, _L)]
                    plsc.store_scatter(tb, [rows_f0[f0], cols], v * scale)

        def start_stores(s, p):
            tb = tbs[p]
            for fb in range(fb_n):
                pltpu.async_copy(tb.at[pl.ds(fb * 8, 8), pl.ds(0, _BB)],
                                 out_hbm.at[s, fb, wid], ssems[p])

        start_gather(0, 0)
        start_gather(1, 1)

        def loop_body(s2, carry):
            for p in range(2):
                s = 2 * s2 + p
                wait_gather(p)

                @pl.when(s2 >= 1)
                def _():
                    wait_stores(p)

                transpose_scale(p)
                start_stores(s, p)

                @pl.when(s + 2 < seq)
                def _():
                    start_gather(s + 2, p)
            return carry

        lax.fori_loop(0, seq // 2, loop_body, 0)
        wait_stores(0)
        wait_stores(1)

    return k


def kernel(token_ids_batch, table):
    batch, seq = token_ids_batch.shape
    vocab, dim = table.shape
    scale = math.sqrt(dim)
    idx_t = jnp.transpose(token_ids_batch).astype(jnp.int32)
    out5d = _build_lookup(batch, seq, dim, scale)(table, idx_t)
    out = jnp.transpose(out5d, (2, 4, 0, 1, 3)).reshape(batch, seq, dim)
    return out
